# baseline (speedup 1.0000x reference)
import functools

import jax
import jax.numpy as jnp
from jax import lax
from jax.experimental import pallas as pl
from jax.experimental.pallas import tpu as pltpu
from jax.experimental.pallas import tpu_sc as plsc

NC = 2
NS = 16
CHUNK = 64


def _make_seg_sum(n_nodes, d, c8, r_proc, acc_rows, zrows, rows_per_tile):
  mesh = plsc.VectorSubcoreMesh(core_axis_name="c", subcore_axis_name="s")

  @functools.partial(
      pl.kernel,
      mesh=mesh,
      out_type=jax.ShapeDtypeStruct((NC, n_nodes, d), jnp.float32),
      scratch_types=[
          pltpu.VMEM((3, 8, CHUNK), jnp.int32),
          pltpu.VMEM((3, 8, CHUNK), jnp.int32),
          pltpu.VMEM((4, CHUNK, d), jnp.float32),
          pltpu.VMEM((16, d), jnp.float32),
          pltpu.VMEM_SHARED((acc_rows, d), jnp.float32),
          [pltpu.SemaphoreType.DMA] * 4,
          [pltpu.SemaphoreType.DMA] * 4,
          pltpu.SemaphoreType.DMA,
          pltpu.SemaphoreType.DMA,
      ],
  )
  def seg(g_hbm, src_hbm, dst_hbm, out_hbm,
          src_v, dst_v, bufs, zbuf, acc, gs, ss, isem, zsem):
    c = lax.axis_index("c")
    s = lax.axis_index("s")
    wid = c * NS + s
    crow0 = c8 * wid
    nc = jnp.clip(r_proc - crow0, 0, c8)
    n_quads = nc // 4
    n_groups = (nc + 7) // 8

    z16 = jnp.zeros((16,), jnp.float32)
    for r in range(16):
      for q in range(d // 16):
        zbuf[r, pl.ds(q * 16, 16)] = z16

    row0 = s * rows_per_tile
    nz = zrows // 16

    def zero_issue(k, carry):
      pltpu.async_copy(zbuf, acc.at[pl.ds(row0 + k * 16, 16)], zsem)
      return carry

    lax.fori_loop(0, nz, zero_issue, 0)

    pltpu.async_copy(src_hbm.at[pl.ds(crow0, 8)], src_v.at[0], isem)
    pltpu.async_copy(dst_hbm.at[pl.ds(crow0, 8)], dst_v.at[0], isem)

    def zero_drain(k, carry):
      pltpu.make_async_copy(zbuf, acc.at[pl.ds(row0, 16)], zsem).wait()
      return carry

    lax.fori_loop(0, nz, zero_drain, 0)
    rem0 = NS * rows_per_tile

    plsc.subcore_barrier()

    def body(i, carry):
      grp = i // 2
      p = grp % 3
      r0 = (4 * i) % 8

      @pl.when(i % 2 == 0)
      def _():
        pltpu.make_async_copy(src_hbm.at[pl.ds(crow0, 8)], src_v.at[p],
                              isem).wait()
        pltpu.make_async_copy(dst_hbm.at[pl.ds(crow0, 8)], dst_v.at[p],
                              isem).wait()
        nxt = jnp.minimum(grp + 1, jnp.maximum(n_groups - 1, 0))
        pn = nxt % 3
        pltpu.async_copy(src_hbm.at[pl.ds(crow0 + nxt * 8, 8)],
                         src_v.at[pn], isem)
        pltpu.async_copy(dst_hbm.at[pl.ds(crow0 + nxt * 8, 8)],
                         dst_v.at[pn], isem)

      @pl.when(i > 0)
      def _():
        for k in range(4):
          pltpu.make_async_copy(bufs.at[k], acc.at[dst_v.at[p, r0 + k]],
                                ss[k]).wait()

      for k in range(4):
        pltpu.async_copy(g_hbm.at[src_v.at[p, r0 + k]], bufs.at[k], gs[k])
      for k in range(4):
        pltpu.make_async_copy(g_hbm.at[src_v.at[p, r0 + k]], bufs.at[k],
                              gs[k]).wait()
        pltpu.async_copy(bufs.at[k], acc.at[dst_v.at[p, r0 + k]], ss[k],
                         add=True)
      return carry

    lax.fori_loop(0, n_quads, body, 0)
    @pl.when(n_quads > 0)
    def _():
      for k in range(4):
        pltpu.make_async_copy(bufs.at[k], acc.at[dst_v.at[0, k]],
                              ss[k]).wait()
    pltpu.make_async_copy(src_hbm.at[pl.ds(crow0, 8)], src_v.at[0],
                          isem).wait()
    pltpu.make_async_copy(dst_hbm.at[pl.ds(crow0, 8)], dst_v.at[0],
                          isem).wait()
    plsc.subcore_barrier()

    pltpu.sync_copy(acc.at[pl.ds(row0, rows_per_tile)],
                    out_hbm.at[c].at[pl.ds(row0, rows_per_tile)])
    rem = n_nodes - NS * rows_per_tile
    if rem:
      @pl.when(s == 0)
      def _():
        pltpu.sync_copy(acc.at[pl.ds(rem0, rem)],
                        out_hbm.at[c].at[pl.ds(rem0, rem)])

  return seg


def _prep_body(e, n, n_spare, br, src_in, dst_in, src_ref, dst_ref):
  i = pl.program_id(0)
  row = i * br + jax.lax.broadcasted_iota(jnp.int32, (br, CHUNK), 0)
  col = jax.lax.broadcasted_iota(jnp.int32, (br, CHUNK), 1)
  eid = row * CHUNK + col
  valid = eid < e
  src_ref[...] = jnp.where(valid, src_in[0], (eid * 41) % n)
  dst_ref[...] = jnp.where(valid, dst_in[0], n + eid % n_spare)


def _prep(edge_index, r_pad, e, n, n_spare):
  br = 16
  if e % CHUNK:
    edge_index = jnp.pad(edge_index, ((0, 0), (0, CHUNK - e % CHUNK)))
  edges3 = edge_index.reshape(2, -1, CHUNK)
  return pl.pallas_call(
      functools.partial(_prep_body, e, n, n_spare, br),
      grid=(r_pad // br,),
      in_specs=[
          pl.BlockSpec((1, br, CHUNK), lambda i: (0, i, 0)),
          pl.BlockSpec((1, br, CHUNK), lambda i: (1, i, 0)),
      ],
      out_specs=[
          pl.BlockSpec((br, CHUNK), lambda i: (i, 0)),
          pl.BlockSpec((br, CHUNK), lambda i: (i, 0)),
      ],
      out_shape=[
          jax.ShapeDtypeStruct((r_pad, CHUNK), jnp.int32),
          jax.ShapeDtypeStruct((r_pad, CHUNK), jnp.int32),
      ],
  )(edges3, edges3)


def _mm_body(x_ref, w_ref, o_ref):
  o_ref[...] = jnp.dot(x_ref[...], w_ref[...],
                       preferred_element_type=jnp.float32)


def _matmul(x, w, blk):
  n, d = x.shape
  return pl.pallas_call(
      _mm_body,
      grid=(n // blk,),
      in_specs=[
          pl.BlockSpec((blk, d), lambda i: (i, 0)),
          pl.BlockSpec((d, w.shape[1]), lambda i: (0, 0)),
      ],
      out_specs=pl.BlockSpec((blk, w.shape[1]), lambda i: (i, 0)),
      out_shape=jax.ShapeDtypeStruct((n, w.shape[1]), jnp.float32),
  )(x, w)


def _fused_body(relu, h_ref, q0_ref, q1_ref, wroot_ref, b_ref, wrel_ref,
                hn_ref, gn_ref):
  t = (q0_ref[0] + q1_ref[0] + b_ref[...]
       + jnp.dot(h_ref[...], wroot_ref[...],
                 preferred_element_type=jnp.float32))
  if relu:
    t = jnp.maximum(t, 0.0)
  hn_ref[...] = t
  gn_ref[...] = jnp.dot(t, wrel_ref[...], preferred_element_type=jnp.float32)


def _fused(h, q, w_root, b, w_rel_next, relu, blk):
  n, d = h.shape
  dn = w_root.shape[1]
  mat = lambda i: (i, 0)
  rep = lambda i: (0, 0)
  return pl.pallas_call(
      functools.partial(_fused_body, relu),
      grid=(n // blk,),
      in_specs=[
          pl.BlockSpec((blk, d), mat),
          pl.BlockSpec((1, blk, dn), lambda i: (0, i, 0)),
          pl.BlockSpec((1, blk, dn), lambda i: (1, i, 0)),
          pl.BlockSpec((d, dn), rep),
          pl.BlockSpec((1, dn), rep),
          pl.BlockSpec((dn, w_rel_next.shape[1]), rep),
      ],
      out_specs=[
          pl.BlockSpec((blk, dn), mat),
          pl.BlockSpec((blk, w_rel_next.shape[1]), mat),
      ],
      out_shape=[
          jax.ShapeDtypeStruct((n, dn), jnp.float32),
          jax.ShapeDtypeStruct((n, w_rel_next.shape[1]), jnp.float32),
      ],
  )(h, q, q, w_root, b.reshape(1, -1), w_rel_next)


def _final_body(h_ref, q0_ref, q1_ref, wroot_ref, b_ref, o_ref):
  o_ref[...] = (q0_ref[0] + q1_ref[0] + b_ref[...]
                + jnp.dot(h_ref[...], wroot_ref[...],
                          preferred_element_type=jnp.float32))


def _final(h, q, w_root, b, blk):
  n, d = h.shape
  dn = w_root.shape[1]
  mat = lambda i: (i, 0)
  rep = lambda i: (0, 0)
  return pl.pallas_call(
      _final_body,
      grid=(n // blk,),
      in_specs=[
          pl.BlockSpec((blk, d), mat),
          pl.BlockSpec((1, blk, dn), lambda i: (0, i, 0)),
          pl.BlockSpec((1, blk, dn), lambda i: (1, i, 0)),
          pl.BlockSpec((d, dn), rep),
          pl.BlockSpec((1, dn), rep),
      ],
      out_specs=pl.BlockSpec((blk, dn), mat),
      out_shape=jax.ShapeDtypeStruct((n, dn), jnp.float32),
  )(h, q, q, w_root, b.reshape(1, -1))


def kernel(x, edge_index, W1_rel, W1_root, b1, W2_rel, W2_root, b2,
           W3_rel, W3_root, b3):
  n, d = x.shape
  e = edge_index.shape[1]
  n_tiles = NC * NS

  n_spare = 240
  r_rows = -(-e // CHUNK)
  r_proc = -(-r_rows // 4) * 4
  per_tile = -(-r_proc // n_tiles)
  c8 = -(-per_tile // 8) * 8
  r_pad = max(r_proc + 8, c8 * (n_tiles - 1) + 8)
  r_pad = -(-r_pad // 16) * 16

  src_p, dst_p = _prep(edge_index.astype(jnp.int32), r_pad, e, n, n_spare)

  rows_per_tile = (n // NS) // 16 * 16
  zrows = -(-(n - (NS - 1) * rows_per_tile) // 16) * 16
  acc_rows = (NS - 1) * rows_per_tile + zrows + n_spare
  seg = _make_seg_sum(n, d, c8, r_proc, acc_rows, zrows, rows_per_tile)

  blk = 1000
  g1 = _matmul(x, W1_rel, blk)
  q1 = seg(g1, src_p, dst_p)
  h1, g2 = _fused(x, q1, W1_root, b1, W2_rel, True, blk)
  q2 = seg(g2, src_p, dst_p)
  h2, g3 = _fused(h1, q2, W2_root, b2, W3_rel, True, blk)
  q3 = seg(g3, src_p, dst_p)
  return _final(h2, q3, W3_root, b3, blk)


# speedup vs baseline: 1.2732x; 1.2732x over previous
import functools

import jax
import jax.numpy as jnp
from jax import lax
from jax.experimental import pallas as pl
from jax.experimental.pallas import tpu as pltpu
from jax.experimental.pallas import tpu_sc as plsc

NC = 2
NS = 16
CHUNK = 64


def _make_seg_sum(n_nodes, d, c8, r_proc, acc_rows, zrows, rows_per_tile):
  mesh = plsc.VectorSubcoreMesh(core_axis_name="c", subcore_axis_name="s")

  @functools.partial(
      pl.kernel,
      mesh=mesh,
      out_type=jax.ShapeDtypeStruct((NC, n_nodes, d), jnp.float32),
      scratch_types=[
          pltpu.VMEM((3, 8, CHUNK), jnp.int32),
          pltpu.VMEM((3, 8, CHUNK), jnp.int32),
          pltpu.VMEM((4, CHUNK, d), jnp.float32),
          pltpu.VMEM((16, d), jnp.float32),
          pltpu.VMEM_SHARED((acc_rows, d), jnp.float32),
          [pltpu.SemaphoreType.DMA] * 4,
          [pltpu.SemaphoreType.DMA] * 4,
          pltpu.SemaphoreType.DMA,
          pltpu.SemaphoreType.DMA,
      ],
  )
  def seg(g_hbm, src_hbm, dst_hbm, out_hbm,
          src_v, dst_v, bufs, zbuf, acc, gs, ss, isem, zsem):
    c = lax.axis_index("c")
    s = lax.axis_index("s")
    wid = c * NS + s
    crow0 = c8 * wid
    nc = jnp.clip(r_proc - crow0, 0, c8)
    n_quads = nc // 4
    n_groups = (nc + 7) // 8

    z16 = jnp.zeros((16,), jnp.float32)
    for r in range(16):
      for q in range(d // 16):
        zbuf[r, pl.ds(q * 16, 16)] = z16

    row0 = s * rows_per_tile
    nz = zrows // 16

    def zero_issue(k, carry):
      pltpu.async_copy(zbuf, acc.at[pl.ds(row0 + k * 16, 16)], zsem)
      return carry

    lax.fori_loop(0, nz, zero_issue, 0)

    pltpu.async_copy(src_hbm.at[pl.ds(crow0, 8)], src_v.at[0], isem)
    pltpu.async_copy(dst_hbm.at[pl.ds(crow0, 8)], dst_v.at[0], isem)

    def zero_drain(k, carry):
      pltpu.make_async_copy(zbuf, acc.at[pl.ds(row0, 16)], zsem).wait()
      return carry

    lax.fori_loop(0, nz, zero_drain, 0)
    rem0 = NS * rows_per_tile

    plsc.subcore_barrier()

    def body(i, carry):
      grp = i // 2
      p = grp % 3
      r0 = (4 * i) % 8

      @pl.when(i % 2 == 0)
      def _():
        pltpu.make_async_copy(src_hbm.at[pl.ds(crow0, 8)], src_v.at[p],
                              isem).wait()
        pltpu.make_async_copy(dst_hbm.at[pl.ds(crow0, 8)], dst_v.at[p],
                              isem).wait()
        nxt = jnp.minimum(grp + 1, jnp.maximum(n_groups - 1, 0))
        pn = nxt % 3
        pltpu.async_copy(src_hbm.at[pl.ds(crow0 + nxt * 8, 8)],
                         src_v.at[pn], isem)
        pltpu.async_copy(dst_hbm.at[pl.ds(crow0 + nxt * 8, 8)],
                         dst_v.at[pn], isem)

      @pl.when(i > 0)
      def _():
        for k in range(4):
          pltpu.make_async_copy(bufs.at[k], acc.at[dst_v.at[p, r0 + k]],
                                ss[k]).wait()

      for k in range(4):
        pltpu.async_copy(g_hbm.at[src_v.at[p, r0 + k]], bufs.at[k], gs[k])
      for k in range(4):
        pltpu.make_async_copy(g_hbm.at[src_v.at[p, r0 + k]], bufs.at[k],
                              gs[k]).wait()
        pltpu.async_copy(bufs.at[k], acc.at[dst_v.at[p, r0 + k]], ss[k],
                         add=True)
      return carry

    lax.fori_loop(0, n_quads, body, 0)
    @pl.when(n_quads > 0)
    def _():
      for k in range(4):
        pltpu.make_async_copy(bufs.at[k], acc.at[dst_v.at[0, k]],
                              ss[k]).wait()
    pltpu.make_async_copy(src_hbm.at[pl.ds(crow0, 8)], src_v.at[0],
                          isem).wait()
    pltpu.make_async_copy(dst_hbm.at[pl.ds(crow0, 8)], dst_v.at[0],
                          isem).wait()
    plsc.subcore_barrier()

    pltpu.sync_copy(acc.at[pl.ds(row0, rows_per_tile)],
                    out_hbm.at[c].at[pl.ds(row0, rows_per_tile)])
    rem = n_nodes - NS * rows_per_tile
    if rem:
      @pl.when(s == 0)
      def _():
        pltpu.sync_copy(acc.at[pl.ds(rem0, rem)],
                        out_hbm.at[c].at[pl.ds(rem0, rem)])

  return seg


def _mm_body(x_ref, w_ref, o_ref):
  o_ref[...] = jnp.dot(x_ref[...], w_ref[...],
                       preferred_element_type=jnp.float32)


def _matmul(x, w, blk):
  n, d = x.shape
  return pl.pallas_call(
      _mm_body,
      grid=(n // blk,),
      in_specs=[
          pl.BlockSpec((blk, d), lambda i: (i, 0)),
          pl.BlockSpec((d, w.shape[1]), lambda i: (0, 0)),
      ],
      out_specs=pl.BlockSpec((blk, w.shape[1]), lambda i: (i, 0)),
      out_shape=jax.ShapeDtypeStruct((n, w.shape[1]), jnp.float32),
  )(x, w)


def _fused_body(relu, h_ref, q0_ref, q1_ref, wroot_ref, b_ref, wrel_ref,
                hn_ref, gn_ref):
  t = (q0_ref[0] + q1_ref[0] + b_ref[...]
       + jnp.dot(h_ref[...], wroot_ref[...],
                 preferred_element_type=jnp.float32))
  if relu:
    t = jnp.maximum(t, 0.0)
  hn_ref[...] = t
  gn_ref[...] = jnp.dot(t, wrel_ref[...], preferred_element_type=jnp.float32)


def _fused(h, q, w_root, b, w_rel_next, relu, blk):
  n, d = h.shape
  dn = w_root.shape[1]
  mat = lambda i: (i, 0)
  rep = lambda i: (0, 0)
  return pl.pallas_call(
      functools.partial(_fused_body, relu),
      grid=(n // blk,),
      in_specs=[
          pl.BlockSpec((blk, d), mat),
          pl.BlockSpec((1, blk, dn), lambda i: (0, i, 0)),
          pl.BlockSpec((1, blk, dn), lambda i: (1, i, 0)),
          pl.BlockSpec((d, dn), rep),
          pl.BlockSpec((1, dn), rep),
          pl.BlockSpec((dn, w_rel_next.shape[1]), rep),
      ],
      out_specs=[
          pl.BlockSpec((blk, dn), mat),
          pl.BlockSpec((blk, w_rel_next.shape[1]), mat),
      ],
      out_shape=[
          jax.ShapeDtypeStruct((n, dn), jnp.float32),
          jax.ShapeDtypeStruct((n, w_rel_next.shape[1]), jnp.float32),
      ],
  )(h, q, q, w_root, b.reshape(1, -1), w_rel_next)


def _final_body(h_ref, q0_ref, q1_ref, wroot_ref, b_ref, o_ref):
  o_ref[...] = (q0_ref[0] + q1_ref[0] + b_ref[...]
                + jnp.dot(h_ref[...], wroot_ref[...],
                          preferred_element_type=jnp.float32))


def _final(h, q, w_root, b, blk):
  n, d = h.shape
  dn = w_root.shape[1]
  mat = lambda i: (i, 0)
  rep = lambda i: (0, 0)
  return pl.pallas_call(
      _final_body,
      grid=(n // blk,),
      in_specs=[
          pl.BlockSpec((blk, d), mat),
          pl.BlockSpec((1, blk, dn), lambda i: (0, i, 0)),
          pl.BlockSpec((1, blk, dn), lambda i: (1, i, 0)),
          pl.BlockSpec((d, dn), rep),
          pl.BlockSpec((1, dn), rep),
      ],
      out_specs=pl.BlockSpec((blk, dn), mat),
      out_shape=jax.ShapeDtypeStruct((n, dn), jnp.float32),
  )(h, q, q, w_root, b.reshape(1, -1))


def kernel(x, edge_index, W1_rel, W1_root, b1, W2_rel, W2_root, b2,
           W3_rel, W3_root, b3):
  n, d = x.shape
  e = edge_index.shape[1]
  n_tiles = NC * NS

  n_spare = 240
  r_rows = -(-e // CHUNK)
  r_proc = -(-r_rows // 4) * 4
  per_tile = -(-r_proc // n_tiles)
  c8 = -(-per_tile // 8) * 8
  r_pad = max(r_proc + 8, c8 * (n_tiles - 1) + 8)
  pad = r_pad * CHUNK - e

  src = edge_index[0].astype(jnp.int32)
  dst = edge_index[1].astype(jnp.int32)
  pad_i = jnp.arange(pad, dtype=jnp.int32)
  src_p = jnp.concatenate([src, (pad_i * 41) % n]).reshape(r_pad, CHUNK)
  dst_p = jnp.concatenate([dst, n + pad_i % n_spare]).reshape(r_pad, CHUNK)

  rows_per_tile = (n // NS) // 16 * 16
  zrows = -(-(n - (NS - 1) * rows_per_tile) // 16) * 16
  acc_rows = (NS - 1) * rows_per_tile + zrows + n_spare
  seg = _make_seg_sum(n, d, c8, r_proc, acc_rows, zrows, rows_per_tile)

  blk = 1000
  g1 = _matmul(x, W1_rel, blk)
  q1 = seg(g1, src_p, dst_p)
  h1, g2 = _fused(x, q1, W1_root, b1, W2_rel, True, blk)
  q2 = seg(g2, src_p, dst_p)
  h2, g3 = _fused(h1, q2, W2_root, b2, W3_rel, True, blk)
  q3 = seg(g3, src_p, dst_p)
  return _final(h2, q3, W3_root, b3, blk)


# speedup vs baseline: 1.2943x; 1.0165x over previous
import functools

import jax
import jax.numpy as jnp
from jax import lax
from jax.experimental import pallas as pl
from jax.experimental.pallas import tpu as pltpu
from jax.experimental.pallas import tpu_sc as plsc

NC = 2
NS = 16
CHUNK = 80


def _make_seg_sum(n_nodes, d, c8, r_proc, acc_rows, zrows, rows_per_tile):
  mesh = plsc.VectorSubcoreMesh(core_axis_name="c", subcore_axis_name="s")

  @functools.partial(
      pl.kernel,
      mesh=mesh,
      out_type=jax.ShapeDtypeStruct((NC, n_nodes, d), jnp.float32),
      scratch_types=[
          pltpu.VMEM((3, 8, CHUNK), jnp.int32),
          pltpu.VMEM((3, 8, CHUNK), jnp.int32),
          pltpu.VMEM((4, CHUNK, d), jnp.float32),
          pltpu.VMEM((16, d), jnp.float32),
          pltpu.VMEM_SHARED((acc_rows, d), jnp.float32),
          [pltpu.SemaphoreType.DMA] * 4,
          [pltpu.SemaphoreType.DMA] * 4,
          pltpu.SemaphoreType.DMA,
          pltpu.SemaphoreType.DMA,
      ],
  )
  def seg(g_hbm, src_hbm, dst_hbm, out_hbm,
          src_v, dst_v, bufs, zbuf, acc, gs, ss, isem, zsem):
    c = lax.axis_index("c")
    s = lax.axis_index("s")
    wid = c * NS + s
    crow0 = c8 * wid
    nc = jnp.clip(r_proc - crow0, 0, c8)
    n_quads = nc // 4
    n_groups = (nc + 7) // 8

    z16 = jnp.zeros((16,), jnp.float32)
    for r in range(16):
      for q in range(d // 16):
        zbuf[r, pl.ds(q * 16, 16)] = z16

    row0 = s * rows_per_tile
    nz = zrows // 16

    def zero_issue(k, carry):
      pltpu.async_copy(zbuf, acc.at[pl.ds(row0 + k * 16, 16)], zsem)
      return carry

    lax.fori_loop(0, nz, zero_issue, 0)

    pltpu.async_copy(src_hbm.at[pl.ds(crow0, 8)], src_v.at[0], isem)
    pltpu.async_copy(dst_hbm.at[pl.ds(crow0, 8)], dst_v.at[0], isem)

    def zero_drain(k, carry):
      pltpu.make_async_copy(zbuf, acc.at[pl.ds(row0, 16)], zsem).wait()
      return carry

    lax.fori_loop(0, nz, zero_drain, 0)
    rem0 = NS * rows_per_tile

    plsc.subcore_barrier()

    def body(i, carry):
      grp = i // 2
      p = grp % 3
      r0 = (4 * i) % 8

      @pl.when(i % 2 == 0)
      def _():
        pltpu.make_async_copy(src_hbm.at[pl.ds(crow0, 8)], src_v.at[p],
                              isem).wait()
        pltpu.make_async_copy(dst_hbm.at[pl.ds(crow0, 8)], dst_v.at[p],
                              isem).wait()
        nxt = jnp.minimum(grp + 1, jnp.maximum(n_groups - 1, 0))
        pn = nxt % 3
        pltpu.async_copy(src_hbm.at[pl.ds(crow0 + nxt * 8, 8)],
                         src_v.at[pn], isem)
        pltpu.async_copy(dst_hbm.at[pl.ds(crow0 + nxt * 8, 8)],
                         dst_v.at[pn], isem)

      @pl.when(i > 0)
      def _():
        for k in range(4):
          pltpu.make_async_copy(bufs.at[k], acc.at[dst_v.at[p, r0 + k]],
                                ss[k]).wait()

      for k in range(4):
        pltpu.async_copy(g_hbm.at[src_v.at[p, r0 + k]], bufs.at[k], gs[k])
      for k in range(4):
        pltpu.make_async_copy(g_hbm.at[src_v.at[p, r0 + k]], bufs.at[k],
                              gs[k]).wait()
        pltpu.async_copy(bufs.at[k], acc.at[dst_v.at[p, r0 + k]], ss[k],
                         add=True)
      return carry

    lax.fori_loop(0, n_quads, body, 0)
    @pl.when(n_quads > 0)
    def _():
      for k in range(4):
        pltpu.make_async_copy(bufs.at[k], acc.at[dst_v.at[0, k]],
                              ss[k]).wait()
    pltpu.make_async_copy(src_hbm.at[pl.ds(crow0, 8)], src_v.at[0],
                          isem).wait()
    pltpu.make_async_copy(dst_hbm.at[pl.ds(crow0, 8)], dst_v.at[0],
                          isem).wait()
    plsc.subcore_barrier()

    pltpu.sync_copy(acc.at[pl.ds(row0, rows_per_tile)],
                    out_hbm.at[c].at[pl.ds(row0, rows_per_tile)])
    rem = n_nodes - NS * rows_per_tile
    if rem:
      @pl.when(s == 0)
      def _():
        pltpu.sync_copy(acc.at[pl.ds(rem0, rem)],
                        out_hbm.at[c].at[pl.ds(rem0, rem)])

  return seg


def _mm_body(x_ref, w_ref, o_ref):
  o_ref[...] = jnp.dot(x_ref[...], w_ref[...],
                       preferred_element_type=jnp.float32)


def _matmul(x, w, blk):
  n, d = x.shape
  return pl.pallas_call(
      _mm_body,
      grid=(n // blk,),
      in_specs=[
          pl.BlockSpec((blk, d), lambda i: (i, 0)),
          pl.BlockSpec((d, w.shape[1]), lambda i: (0, 0)),
      ],
      out_specs=pl.BlockSpec((blk, w.shape[1]), lambda i: (i, 0)),
      out_shape=jax.ShapeDtypeStruct((n, w.shape[1]), jnp.float32),
  )(x, w)


def _fused_body(relu, h_ref, q0_ref, q1_ref, wroot_ref, b_ref, wrel_ref,
                hn_ref, gn_ref):
  t = (q0_ref[0] + q1_ref[0] + b_ref[...]
       + jnp.dot(h_ref[...], wroot_ref[...],
                 preferred_element_type=jnp.float32))
  if relu:
    t = jnp.maximum(t, 0.0)
  hn_ref[...] = t
  gn_ref[...] = jnp.dot(t, wrel_ref[...], preferred_element_type=jnp.float32)


def _fused(h, q, w_root, b, w_rel_next, relu, blk):
  n, d = h.shape
  dn = w_root.shape[1]
  mat = lambda i: (i, 0)
  rep = lambda i: (0, 0)
  return pl.pallas_call(
      functools.partial(_fused_body, relu),
      grid=(n // blk,),
      in_specs=[
          pl.BlockSpec((blk, d), mat),
          pl.BlockSpec((1, blk, dn), lambda i: (0, i, 0)),
          pl.BlockSpec((1, blk, dn), lambda i: (1, i, 0)),
          pl.BlockSpec((d, dn), rep),
          pl.BlockSpec((1, dn), rep),
          pl.BlockSpec((dn, w_rel_next.shape[1]), rep),
      ],
      out_specs=[
          pl.BlockSpec((blk, dn), mat),
          pl.BlockSpec((blk, w_rel_next.shape[1]), mat),
      ],
      out_shape=[
          jax.ShapeDtypeStruct((n, dn), jnp.float32),
          jax.ShapeDtypeStruct((n, w_rel_next.shape[1]), jnp.float32),
      ],
  )(h, q, q, w_root, b.reshape(1, -1), w_rel_next)


def _final_body(h_ref, q0_ref, q1_ref, wroot_ref, b_ref, o_ref):
  o_ref[...] = (q0_ref[0] + q1_ref[0] + b_ref[...]
                + jnp.dot(h_ref[...], wroot_ref[...],
                          preferred_element_type=jnp.float32))


def _final(h, q, w_root, b, blk):
  n, d = h.shape
  dn = w_root.shape[1]
  mat = lambda i: (i, 0)
  rep = lambda i: (0, 0)
  return pl.pallas_call(
      _final_body,
      grid=(n // blk,),
      in_specs=[
          pl.BlockSpec((blk, d), mat),
          pl.BlockSpec((1, blk, dn), lambda i: (0, i, 0)),
          pl.BlockSpec((1, blk, dn), lambda i: (1, i, 0)),
          pl.BlockSpec((d, dn), rep),
          pl.BlockSpec((1, dn), rep),
      ],
      out_specs=pl.BlockSpec((blk, dn), mat),
      out_shape=jax.ShapeDtypeStruct((n, dn), jnp.float32),
  )(h, q, q, w_root, b.reshape(1, -1))


def kernel(x, edge_index, W1_rel, W1_root, b1, W2_rel, W2_root, b2,
           W3_rel, W3_root, b3):
  n, d = x.shape
  e = edge_index.shape[1]
  n_tiles = NC * NS

  n_spare = 240
  r_rows = -(-e // CHUNK)
  r_proc = -(-r_rows // 4) * 4
  per_tile = -(-r_proc // n_tiles)
  c8 = -(-per_tile // 8) * 8
  r_pad = max(r_proc + 8, c8 * (n_tiles - 1) + 8)
  pad = r_pad * CHUNK - e

  src = edge_index[0].astype(jnp.int32)
  dst = edge_index[1].astype(jnp.int32)
  pad_i = jnp.arange(pad, dtype=jnp.int32)
  src_p = jnp.concatenate([src, (pad_i * 41) % n]).reshape(r_pad, CHUNK)
  dst_p = jnp.concatenate([dst, n + pad_i % n_spare]).reshape(r_pad, CHUNK)

  rows_per_tile = (n // NS) // 16 * 16
  zrows = -(-(n - (NS - 1) * rows_per_tile) // 16) * 16
  acc_rows = (NS - 1) * rows_per_tile + zrows + n_spare
  seg = _make_seg_sum(n, d, c8, r_proc, acc_rows, zrows, rows_per_tile)

  blk = 1000
  g1 = _matmul(x, W1_rel, blk)
  q1 = seg(g1, src_p, dst_p)
  h1, g2 = _fused(x, q1, W1_root, b1, W2_rel, True, blk)
  q2 = seg(g2, src_p, dst_p)
  h2, g3 = _fused(h1, q2, W2_root, b2, W3_rel, True, blk)
  q3 = seg(g3, src_p, dst_p)
  return _final(h2, q3, W3_root, b3, blk)
